# Initial kernel scaffold; baseline (speedup 1.0000x reference)
import jax
import jax.numpy as jnp
from jax.experimental import pallas as pl


def kernel(z, embedding_weight):
    raise NotImplementedError("write your pallas kernel here")


# speedup vs baseline: 1.0860x; 1.0860x over previous
import functools

import jax
import jax.numpy as jnp
from jax import lax
from jax.experimental import pallas as pl
from jax.experimental.pallas import tpu as pltpu
from jax.experimental.pallas import tpu_sc as plsc

B = 8
C = 256
HW = 1024
V = 8192
CB = 1024
NCB = V // CB

EPS = 1e-12
BIG_I32 = 2**31 - 1


def _enorm_kernel(e_ref, nrm_ref, enorm_ref):
    enorm_ref[...] = e_ref[...] * pl.reciprocal(nrm_ref[...], approx=True)


def _normalize_codebook(embedding_weight, norm_e):
    return pl.pallas_call(
        _enorm_kernel,
        grid=(NCB,),
        in_specs=[
            pl.BlockSpec((CB, C), lambda c: (c, 0)),
            pl.BlockSpec((CB, 1), lambda c: (c, 0)),
        ],
        out_specs=pl.BlockSpec((CB, C), lambda c: (c, 0)),
        out_shape=jax.ShapeDtypeStruct((V, C), jnp.float32),
    )(embedding_weight, norm_e)


def _argmin_kernel(z_ref, znrm_ref, en_ref, idx_ref):
    z_n = z_ref[0] * pl.reciprocal(znrm_ref[0], approx=True)
    zn2 = jnp.sum(z_n * z_n, axis=0, keepdims=True)

    def body(c, carry):
        best_d, best_i = carry
        e_n = en_ref[pl.ds(c * CB, CB), :]
        en2 = jnp.sum(e_n * e_n, axis=1, keepdims=True)
        s = jax.lax.dot_general(
            e_n, z_n, (((1,), (0,)), ((), ())),
            preferred_element_type=jnp.float32)
        d = (zn2 + en2) - 2.0 * s
        m = jnp.min(d, axis=0, keepdims=True)
        rows = lax.broadcasted_iota(jnp.int32, (CB, HW), 0) + c * CB
        bidx = jnp.min(jnp.where(d == m, rows, BIG_I32), axis=0,
                       keepdims=True)
        better = m < best_d
        return (jnp.where(better, m, best_d),
                jnp.where(better, bidx, best_i))

    init = (jnp.full((1, HW), jnp.inf, jnp.float32),
            jnp.zeros((1, HW), jnp.int32))
    _, best_i = lax.fori_loop(0, NCB, body, init)
    idx_ref[0] = best_i


def _vq_distance_argmin(z_r, norm_z, e_norm):
    return pl.pallas_call(
        _argmin_kernel,
        grid=(B,),
        in_specs=[
            pl.BlockSpec((1, C, HW), lambda b: (b, 0, 0)),
            pl.BlockSpec((1, 1, HW), lambda b: (b, 0, 0)),
            pl.BlockSpec((V, C), lambda b: (0, 0)),
        ],
        out_specs=pl.BlockSpec((1, 1, HW), lambda b: (b, 0, 0)),
        out_shape=jax.ShapeDtypeStruct((B, 1, HW), jnp.int32),
    )(z_r, norm_z, e_norm)


_SC_NC = 2
_SC_NS = 16
_SC_NW = _SC_NC * _SC_NS
_NTOK = B * HW
_B_PER_W = _NTOK // _SC_NW
_GCHUNK = 128
_NCHUNK = _B_PER_W // _GCHUNK


def _sc_gather(table, idx):
    mesh = plsc.VectorSubcoreMesh(core_axis_name="c", subcore_axis_name="s")

    @functools.partial(
        pl.kernel,
        mesh=mesh,
        out_type=jax.ShapeDtypeStruct((_NTOK, C), jnp.float32),
        scratch_types=[
            pltpu.VMEM((_NCHUNK, _GCHUNK), jnp.int32),
            pltpu.VMEM((_B_PER_W, C), jnp.float32),
            pltpu.SemaphoreType.DMA,
        ],
    )
    def k(table_hbm, idx_hbm, out_hbm, idx_v, rows_v, sem):
        wid = lax.axis_index("s") * _SC_NC + lax.axis_index("c")
        base = wid * _B_PER_W
        copies = []
        for j in range(_NCHUNK):
            pltpu.sync_copy(idx_hbm.at[pl.ds(base + j * _GCHUNK, _GCHUNK)],
                            idx_v.at[j])
            copies.append(pltpu.async_copy(
                table_hbm.at[idx_v.at[j]],
                rows_v.at[pl.ds(j * _GCHUNK, _GCHUNK)], sem))
        for cp in copies:
            cp.wait()
        pltpu.sync_copy(rows_v, out_hbm.at[pl.ds(base, _B_PER_W)])

    return k(table, idx)


def kernel(z, embedding_weight):
    batch, channels, height, width = z.shape
    hw = height * width

    norm_e = jnp.maximum(
        jnp.linalg.norm(embedding_weight, ord=2, axis=-1, keepdims=True), EPS)
    zf = z.transpose(0, 2, 3, 1).reshape(-1, channels)
    norm_z = jnp.maximum(
        jnp.linalg.norm(zf, ord=2, axis=-1, keepdims=True), EPS)
    norm_z = norm_z.reshape(batch, 1, hw)

    z_r = z.reshape(batch, channels, hw)
    e_norm = _normalize_codebook(embedding_weight, norm_e)
    idx3 = _vq_distance_argmin(z_r, norm_z, e_norm)
    idx_flat = idx3.reshape(-1)
    z_q_rows = _sc_gather(e_norm, idx_flat)
    z_q = z_q_rows.reshape(batch, height, width, channels).transpose(0, 3, 1, 2)
    indices = idx3.reshape(batch, height, width)
    return (z_q, indices)
